# baseline (speedup 1.0000x reference)
import functools

import jax
import jax.numpy as jnp
from jax import lax
from jax.experimental import pallas as pl
from jax.experimental.pallas import tpu as pltpu
from jax.experimental.pallas import tpu_sc as plsc

N = 10000
E = 320000
HEADS = 8
MID = 8
OUT = 64
IN = 128
SLOPE = 0.28

NC = 2
NS = 16
NW = NC * NS
L = 16

NP = 10112
EP = 327680
CA = 128
CB2 = 64
RPT = NP // NS

NORTH = 1
ETN = 13312
ETS = 7168
NT = NS * ETN

_f32 = jnp.float32

_GDN = lax.GatherDimensionNumbers(
    offset_dims=(), collapsed_slice_dims=(0,), start_index_map=(0,))


def _lgather(v, idx):
  return lax.gather(v, idx.reshape(L, 1), _GDN, (1,),
                    mode=lax.GatherScatterMode.PROMISE_IN_BOUNDS)


def _bcast(v, lane):
  return _lgather(v, jnp.full((L,), lane, jnp.int32))


def _sc_mesh():
  return plsc.VectorSubcoreMesh(
      core_axis_name="c", subcore_axis_name="s", num_cores=NC,
      num_subcores=NS)


def _passA_body(src_h, dst_h, as_h, ad_h, g_h, z16_h, ea_h, den_h,
                idxs, idxd, is0, id0, is1, id1, ids0, ids1,
                asb0, adb0, asb1, adb1, eab0, eab1, gbuf,
                gsem0, gsem1, wsem0, wsem1, den_sp):
  cid = lax.axis_index("c")
  sid = lax.axis_index("s")
  row0 = pl.multiple_of(sid * RPT, 8)
  for i in range(RPT // CA):
    pltpu.sync_copy(z16_h, den_sp.at[pl.ds(row0 + i * CA, CA)])
  pltpu.sync_copy(z16_h.at[pl.ds(0, RPT % CA)],
                  den_sp.at[pl.ds(row0 + (RPT // CA) * CA, RPT % CA)])
  pltpu.sync_copy(g_h, gbuf)
  plsc.subcore_barrier()
  gv = gbuf[...]

  def ldidx(src2d, k, dst1d):
    for j in range(CA // L):
      dst1d[pl.ds(j * L, L)] = src2d[k * (CA // L) + j]

  def pipe(core_base, ett):
    er = ett // L
    rb = pl.multiple_of((core_base + sid * ett) // L, 8)
    pltpu.sync_copy(src_h.at[pl.ds(rb, er)], idxs.at[pl.ds(0, er)])
    pltpu.sync_copy(dst_h.at[pl.ds(rb, er)], idxd.at[pl.ds(0, er)])
    base0 = core_base + sid * ett

    def issue(k, isb, idb_, asb, adb, sem):
      ldidx(idxs, k, isb)
      ldidx(idxd, k, idb_)
      pltpu.async_copy(as_h.at[isb], asb, sem)
      pltpu.async_copy(ad_h.at[idb_], adb, sem)

    def wait_g(isb, idb_, asb, adb, sem):
      pltpu.make_async_copy(as_h.at[isb], asb, sem).wait()
      pltpu.make_async_copy(ad_h.at[idb_], adb, sem).wait()

    def compute(k, sidb, asb, adb, eab, wsem):
      def edge(e, cc):
        v = asb[e] + adb[e]
        a = jnp.maximum(v, v * SLOPE) - gv
        eab[e] = jnp.exp(a)
        return cc
      lax.fori_loop(0, CA, edge, 0, unroll=4)
      base = pl.multiple_of(base0 + k * CA, CA)
      pltpu.async_copy(eab, ea_h.at[pl.ds(base, CA)], wsem)
      ldidx(idxd, k, sidb)
      pltpu.sync_copy(eab, den_sp.at[sidb], add=True)

    def wait_w(eab, wsem):
      pltpu.make_async_copy(eab, ea_h.at[pl.ds(0, CA)], wsem).wait()

    nh = ett // CA // 2
    issue(0, is0, id0, asb0, adb0, gsem0)

    def body(k2, cc):
      k = 2 * k2
      issue(k + 1, is1, id1, asb1, adb1, gsem1)
      wait_g(is0, id0, asb0, adb0, gsem0)

      @pl.when(k2 > 0)
      def _():
        wait_w(eab0, wsem0)
      compute(k, ids0, asb0, adb0, eab0, wsem0)

      @pl.when(k2 < nh - 1)
      def _():
        issue(k + 2, is0, id0, asb0, adb0, gsem0)
      wait_g(is1, id1, asb1, adb1, gsem1)

      @pl.when(k2 > 0)
      def _():
        wait_w(eab1, wsem1)
      compute(k + 1, ids1, asb1, adb1, eab1, wsem1)
      return cc

    lax.fori_loop(0, nh, body, 0)
    wait_w(eab0, wsem0)
    wait_w(eab1, wsem1)

  @pl.when(cid == NORTH)
  def _():
    pipe(0, ETN)

  @pl.when(cid != NORTH)
  def _():
    pipe(NT, ETS)

  plsc.subcore_barrier()
  pltpu.sync_copy(den_sp.at[pl.ds(row0, RPT)],
                  den_h.at[cid, pl.ds(row0, RPT)])


_passA = functools.partial(
    pl.kernel, _passA_body,
    out_type=[jax.ShapeDtypeStruct((EP, L), _f32),
              jax.ShapeDtypeStruct((NC, NP, L), _f32)],
    mesh=_sc_mesh(),
    compiler_params=pltpu.CompilerParams(use_tc_tiling_on_sc=False),
    scratch_types=[pltpu.VMEM((ETN // L, L), jnp.int32),
                   pltpu.VMEM((ETN // L, L), jnp.int32),
                   pltpu.VMEM((CA,), jnp.int32),
                   pltpu.VMEM((CA,), jnp.int32),
                   pltpu.VMEM((CA,), jnp.int32),
                   pltpu.VMEM((CA,), jnp.int32),
                   pltpu.VMEM((CA,), jnp.int32),
                   pltpu.VMEM((CA,), jnp.int32),
                   pltpu.VMEM((CA, L), _f32),
                   pltpu.VMEM((CA, L), _f32),
                   pltpu.VMEM((CA, L), _f32),
                   pltpu.VMEM((CA, L), _f32),
                   pltpu.VMEM((CA, L), _f32),
                   pltpu.VMEM((CA, L), _f32),
                   pltpu.VMEM((L,), _f32),
                   pltpu.SemaphoreType.DMA,
                   pltpu.SemaphoreType.DMA,
                   pltpu.SemaphoreType.DMA,
                   pltpu.SemaphoreType.DMA,
                   pltpu.VMEM_SHARED((NP, L), _f32)])


def _passB1_body(src_h, dst_h, ea_h, inv_h, htab_h, z4_h, o_h,
                 idxs, idxd, is0, id0, is1, id1, ids0, ids1,
                 hb0, idb0, eab0, hb1, idb1, eab1, msg0, msg1,
                 gsem0, gsem1, ssem0, ssem1, out_sp):
  cid = lax.axis_index("c")
  sid = lax.axis_index("s")
  row0 = pl.multiple_of(sid * RPT, 8)
  for i in range(RPT // CA):
    pltpu.sync_copy(z4_h, out_sp.at[pl.ds(row0 + i * CA, CA)])
  pltpu.sync_copy(z4_h.at[pl.ds(0, RPT % CA)],
                  out_sp.at[pl.ds(row0 + (RPT // CA) * CA, RPT % CA)])
  plsc.subcore_barrier()
  half = lax.shift_right_logical(
      lax.broadcasted_iota(jnp.int32, (L,), 0), 3)

  def ldidx(src2d, k, dst1d):
    for j in range(CA // L):
      dst1d[pl.ds(j * L, L)] = src2d[k * (CA // L) + j]

  def pipe(core_base, ett):
    er = ett // L
    rb = pl.multiple_of((core_base + sid * ett) // L, 8)
    pltpu.sync_copy(src_h.at[pl.ds(rb, er)], idxs.at[pl.ds(0, er)])
    pltpu.sync_copy(dst_h.at[pl.ds(rb, er)], idxd.at[pl.ds(0, er)])
    base0 = core_base + sid * ett

    def issue(k, isb, idb_, hb, idb, eab, sem):
      ldidx(idxs, k, isb)
      ldidx(idxd, k, idb_)
      pltpu.async_copy(htab_h.at[isb], hb, sem)
      pltpu.async_copy(inv_h.at[idb_], idb, sem)
      base = pl.multiple_of(base0 + k * CA, CA)
      pltpu.async_copy(ea_h.at[pl.ds(base, CA)], eab, sem)

    def wait_g(isb, idb_, hb, idb, eab, sem):
      pltpu.make_async_copy(htab_h.at[isb], hb, sem).wait()
      pltpu.make_async_copy(inv_h.at[idb_], idb, sem).wait()
      pltpu.make_async_copy(ea_h.at[pl.ds(0, CA)], eab, sem).wait()

    def compute(k, sidb, hb, idb, eab, msg, ssem):
      ldidx(idxd, k, sidb)

      def edge(e, cc):
        cv = eab[e] * idb[e]
        for j in range(4):
          cj = _lgather(cv, 2 * j + half)
          msg[e, j] = hb[e, j] * cj
        return cc
      lax.fori_loop(0, CA, edge, 0, unroll=4)
      pltpu.async_copy(msg, out_sp.at[sidb], ssem, add=True)

    def wait_s(sidb, msg, ssem):
      pltpu.make_async_copy(msg, out_sp.at[sidb], ssem).wait()

    nh = ett // CA // 2
    issue(0, is0, id0, hb0, idb0, eab0, gsem0)

    def body(k2, cc):
      k = 2 * k2
      issue(k + 1, is1, id1, hb1, idb1, eab1, gsem1)
      wait_g(is0, id0, hb0, idb0, eab0, gsem0)

      @pl.when(k2 > 0)
      def _():
        wait_s(ids0, msg0, ssem0)
      compute(k, ids0, hb0, idb0, eab0, msg0, ssem0)

      @pl.when(k2 < nh - 1)
      def _():
        issue(k + 2, is0, id0, hb0, idb0, eab0, gsem0)
      wait_g(is1, id1, hb1, idb1, eab1, gsem1)

      @pl.when(k2 > 0)
      def _():
        wait_s(ids1, msg1, ssem1)
      compute(k + 1, ids1, hb1, idb1, eab1, msg1, ssem1)
      return cc

    lax.fori_loop(0, nh, body, 0)
    wait_s(ids0, msg0, ssem0)
    wait_s(ids1, msg1, ssem1)

  @pl.when(cid == NORTH)
  def _():
    pipe(0, ETN)

  @pl.when(cid != NORTH)
  def _():
    pipe(NT, ETS)

  plsc.subcore_barrier()
  pltpu.sync_copy(out_sp.at[pl.ds(row0, RPT)],
                  o_h.at[cid, pl.ds(row0, RPT)])


_passB1 = functools.partial(
    pl.kernel, _passB1_body,
    out_type=[jax.ShapeDtypeStruct((NC, NP, 4, L), _f32)],
    mesh=_sc_mesh(),
    compiler_params=pltpu.CompilerParams(use_tc_tiling_on_sc=False),
    scratch_types=[pltpu.VMEM((ETN // L, L), jnp.int32),
                   pltpu.VMEM((ETN // L, L), jnp.int32),
                   pltpu.VMEM((CA,), jnp.int32),
                   pltpu.VMEM((CA,), jnp.int32),
                   pltpu.VMEM((CA,), jnp.int32),
                   pltpu.VMEM((CA,), jnp.int32),
                   pltpu.VMEM((CA,), jnp.int32),
                   pltpu.VMEM((CA,), jnp.int32),
                   pltpu.VMEM((CA, 4, L), _f32),
                   pltpu.VMEM((CA, L), _f32),
                   pltpu.VMEM((CA, L), _f32),
                   pltpu.VMEM((CA, 4, L), _f32),
                   pltpu.VMEM((CA, L), _f32),
                   pltpu.VMEM((CA, L), _f32),
                   pltpu.VMEM((CA, 4, L), _f32),
                   pltpu.VMEM((CA, 4, L), _f32),
                   pltpu.SemaphoreType.DMA,
                   pltpu.SemaphoreType.DMA,
                   pltpu.SemaphoreType.DMA,
                   pltpu.SemaphoreType.DMA,
                   pltpu.VMEM_SHARED((NP, 4, L), _f32)])


def _passB2_body(src_h, dst_h, ea_h, inv_h, htab_h, z4_h, o_h,
                 is0, id0, is1, id1, ids0, ids1,
                 hb0, idb0, eab0, hb1, idb1, eab1, msg0, msg1,
                 gsem0, gsem1, ssem0, ssem1, isem0, isem1, out_sp):
  cid = lax.axis_index("c")
  sid = lax.axis_index("s")
  row0 = pl.multiple_of(sid * RPT, 8)
  for i in range(RPT // CA):
    pltpu.sync_copy(z4_h, out_sp.at[pl.ds(row0 + i * CA, CA)])
  pltpu.sync_copy(z4_h.at[pl.ds(0, RPT % CA)],
                  out_sp.at[pl.ds(row0 + (RPT // CA) * CA, RPT % CA)])
  plsc.subcore_barrier()

  def cpidx(idb_, sidb):
    for j in range(CB2 // L):
      sidb[pl.ds(j * L, L)] = idb_[pl.ds(j * L, L)]

  def pipe(core_base, ett):
    base0 = core_base + sid * ett

    def issue_idx(k, isb, idb_, isem):
      base = pl.multiple_of(base0 + k * CB2, CB2)
      pltpu.async_copy(src_h.at[pl.ds(base, CB2)], isb, isem)
      pltpu.async_copy(dst_h.at[pl.ds(base, CB2)], idb_, isem)

    def wait_idx(isb, idb_, isem):
      pltpu.make_async_copy(src_h.at[pl.ds(0, CB2)], isb, isem).wait()
      pltpu.make_async_copy(dst_h.at[pl.ds(0, CB2)], idb_, isem).wait()

    def issue_g(k, isb, idb_, hb, idb, eab, sem):
      pltpu.async_copy(htab_h.at[isb], hb, sem)
      pltpu.async_copy(inv_h.at[idb_], idb, sem)
      base = pl.multiple_of(base0 + k * CB2, CB2)
      pltpu.async_copy(ea_h.at[pl.ds(base, CB2)], eab, sem)

    def wait_g(isb, idb_, hb, idb, eab, sem):
      pltpu.make_async_copy(htab_h.at[isb], hb, sem).wait()
      pltpu.make_async_copy(inv_h.at[idb_], idb, sem).wait()
      pltpu.make_async_copy(ea_h.at[pl.ds(0, CB2)], eab, sem).wait()

    def compute(hb, idb, eab, msg, sidb, ssem):
      def edge(e, cc):
        cv = eab[e] * idb[e]
        acc = [None] * 4
        for h in range(HEADS):
          cf = _bcast(cv, h)
          for j in range(4):
            t = hb[e, h * 4 + j] * cf
            acc[j] = t if acc[j] is None else acc[j] + t
        for j in range(4):
          msg[e, j] = acc[j]
        return cc
      lax.fori_loop(0, CB2, edge, 0, unroll=2)
      pltpu.async_copy(msg, out_sp.at[sidb], ssem, add=True)

    def wait_s(sidb, msg, ssem):
      pltpu.make_async_copy(msg, out_sp.at[sidb], ssem).wait()

    nh = ett // CB2 // 2
    issue_idx(0, is0, id0, isem0)
    issue_idx(1, is1, id1, isem1)
    wait_idx(is0, id0, isem0)
    issue_g(0, is0, id0, hb0, idb0, eab0, gsem0)

    def body(k2, cc):
      k = 2 * k2
      wait_idx(is1, id1, isem1)
      issue_g(k + 1, is1, id1, hb1, idb1, eab1, gsem1)

      wait_g(is0, id0, hb0, idb0, eab0, gsem0)

      @pl.when(k2 > 0)
      def _():
        wait_s(ids0, msg0, ssem0)
      cpidx(id0, ids0)

      @pl.when(k2 < nh - 1)
      def _():
        issue_idx(k + 2, is0, id0, isem0)
      compute(hb0, idb0, eab0, msg0, ids0, ssem0)

      @pl.when(k2 < nh - 1)
      def _():
        wait_idx(is0, id0, isem0)
        issue_g(k + 2, is0, id0, hb0, idb0, eab0, gsem0)

      wait_g(is1, id1, hb1, idb1, eab1, gsem1)

      @pl.when(k2 > 0)
      def _():
        wait_s(ids1, msg1, ssem1)
      cpidx(id1, ids1)

      @pl.when(k2 < nh - 1)
      def _():
        issue_idx(k + 3, is1, id1, isem1)
      compute(hb1, idb1, eab1, msg1, ids1, ssem1)
      return cc

    lax.fori_loop(0, nh, body, 0)
    wait_s(ids0, msg0, ssem0)
    wait_s(ids1, msg1, ssem1)

  @pl.when(cid == NORTH)
  def _():
    pipe(0, ETN)

  @pl.when(cid != NORTH)
  def _():
    pipe(NT, ETS)

  plsc.subcore_barrier()
  pltpu.sync_copy(out_sp.at[pl.ds(row0, RPT)],
                  o_h.at[cid, pl.ds(row0, RPT)])


_passB2 = functools.partial(
    pl.kernel, _passB2_body,
    out_type=[jax.ShapeDtypeStruct((NC, NP, 4, L), _f32)],
    mesh=_sc_mesh(),
    compiler_params=pltpu.CompilerParams(use_tc_tiling_on_sc=False),
    scratch_types=[pltpu.VMEM((CB2,), jnp.int32),
                   pltpu.VMEM((CB2,), jnp.int32),
                   pltpu.VMEM((CB2,), jnp.int32),
                   pltpu.VMEM((CB2,), jnp.int32),
                   pltpu.VMEM((CB2,), jnp.int32),
                   pltpu.VMEM((CB2,), jnp.int32),
                   pltpu.VMEM((CB2, 32, L), _f32),
                   pltpu.VMEM((CB2, L), _f32),
                   pltpu.VMEM((CB2, L), _f32),
                   pltpu.VMEM((CB2, 32, L), _f32),
                   pltpu.VMEM((CB2, L), _f32),
                   pltpu.VMEM((CB2, L), _f32),
                   pltpu.VMEM((CB2, 4, L), _f32),
                   pltpu.VMEM((CB2, 4, L), _f32),
                   pltpu.SemaphoreType.DMA,
                   pltpu.SemaphoreType.DMA,
                   pltpu.SemaphoreType.DMA,
                   pltpu.SemaphoreType.DMA,
                   pltpu.SemaphoreType.DMA,
                   pltpu.SemaphoreType.DMA,
                   pltpu.VMEM_SHARED((NP, 4, L), _f32)])


def _tc1_body(xp_ref, w_ref, h_ref, as_ref, ad_ref, g_ref):
  hcat = jnp.dot(xp_ref[...], w_ref[...], preferred_element_type=_f32)
  h_ref[...] = hcat[:, :64]
  a_s = hcat[:, 64:80]
  a_d = hcat[:, 80:96]
  as_ref[...] = a_s
  ad_ref[...] = a_d
  g_ref[...] = (jnp.max(a_s, axis=0) + jnp.max(a_d, axis=0)).reshape(1, L)


def _dsum_body(scale, den_ref, inv_ref):
  d = den_ref[0] + den_ref[1]
  inv_ref[...] = scale / (d + 1e-16)


def _tc2_body(o_ref, b1_ref, w2_ref, as2_ref, ad2_ref,
              h2_ref, a2s_ref, a2d_ref, g2_ref):
  z = o_ref[0] + o_ref[1] + b1_ref[...]
  z = jnp.where(z > 0, z, jnp.exp(jnp.minimum(z, 0.0)) - 1.0)
  ri = lax.broadcasted_iota(jnp.int32, (NP, 64), 0)
  z = jnp.where(ri < N, z, 0.0)
  h2 = jnp.dot(z, w2_ref[...], preferred_element_type=_f32)
  h2_ref[...] = h2
  a2s = jnp.dot(h2, as2_ref[...], preferred_element_type=_f32)
  a2d = jnp.dot(h2, ad2_ref[...], preferred_element_type=_f32)
  a2s_ref[...] = a2s
  a2d_ref[...] = a2d
  g2_ref[...] = (jnp.max(a2s, axis=0) + jnp.max(a2d, axis=0)).reshape(1, L)


def _tc3_body(o_ref, b2_ref, out_ref):
  o = o_ref[0] + o_ref[1] + b2_ref[...]
  ri = lax.broadcasted_iota(jnp.int32, (NP, OUT), 0)
  o = jnp.where(ri < N, o, 0.0)
  m = jnp.max(o, axis=1, keepdims=True)
  o = o - m
  out_ref[...] = o - jnp.log(jnp.sum(jnp.exp(o), axis=1, keepdims=True))


def kernel(node_feature, adj_list, W1, att_src1, att_dst1, b1,
           W2, att_src2, att_dst2, b2):
  W1r = W1.reshape(IN, HEADS, MID)
  AS1 = jnp.pad(jnp.einsum("ihm,hm->ih", W1r, att_src1),
                ((0, 0), (0, L - HEADS)))
  AD1 = jnp.pad(jnp.einsum("ihm,hm->ih", W1r, att_dst1),
                ((0, 0), (0, L - HEADS)))
  Wcat = jnp.concatenate([W1, AS1, AD1], axis=1)

  eye8 = jnp.eye(HEADS, dtype=_f32)
  AS2 = jnp.pad((att_src2[:, :, None] * eye8[:, None, :])
                .reshape(HEADS * OUT, HEADS), ((0, 0), (0, L - HEADS)))
  AD2 = jnp.pad((att_dst2[:, :, None] * eye8[:, None, :])
                .reshape(HEADS * OUT, HEADS), ((0, 0), (0, L - HEADS)))

  xp = jnp.pad(node_feature, ((0, NP - N), (0, 0)))
  pad_e = jnp.full((EP - E,), N, jnp.int32)
  srcp = jnp.concatenate([adj_list[0].astype(jnp.int32), pad_e])
  dstp = jnp.concatenate([adj_list[1].astype(jnp.int32), pad_e])
  srcA = srcp.reshape(EP // L, L)
  dstA = dstp.reshape(EP // L, L)

  z16 = jnp.zeros((CA, L), _f32)
  z4 = jnp.zeros((CA, 4, L), _f32)

  h1t, a1s, a1d, g1 = pl.pallas_call(
      _tc1_body,
      out_shape=[jax.ShapeDtypeStruct((NP, 64), _f32),
                 jax.ShapeDtypeStruct((NP, L), _f32),
                 jax.ShapeDtypeStruct((NP, L), _f32),
                 jax.ShapeDtypeStruct((1, L), _f32)])(xp, Wcat)

  ea1, den1 = _passA()(srcA, dstA, a1s, a1d, g1.reshape(L), z16)
  inv1 = pl.pallas_call(
      functools.partial(_dsum_body, 1.0),
      out_shape=jax.ShapeDtypeStruct((NP, L), _f32))(den1)
  (o1,) = _passB1()(srcA, dstA, ea1, inv1, h1t.reshape(NP, 4, L), z4)

  h2t, a2s, a2d, g2 = pl.pallas_call(
      _tc2_body,
      out_shape=[jax.ShapeDtypeStruct((NP, HEADS * OUT), _f32),
                 jax.ShapeDtypeStruct((NP, L), _f32),
                 jax.ShapeDtypeStruct((NP, L), _f32),
                 jax.ShapeDtypeStruct((1, L), _f32)])(
                     o1.reshape(NC, NP, 64), b1.reshape(1, 64), W2, AS2, AD2)

  ea2, den2 = _passA()(srcA, dstA, a2s, a2d, g2.reshape(L), z16)
  inv2 = pl.pallas_call(
      functools.partial(_dsum_body, 1.0 / HEADS),
      out_shape=jax.ShapeDtypeStruct((NP, L), _f32))(den2)
  (o2,) = _passB2()(srcp, dstp, ea2, inv2, h2t.reshape(NP, 32, L), z4)

  outp = pl.pallas_call(
      _tc3_body,
      out_shape=jax.ShapeDtypeStruct((NP, OUT), _f32))(
          o2.reshape(NC, NP, OUT), b2.reshape(1, OUT))
  return outp[:N]


# speedup vs baseline: 1.0718x; 1.0718x over previous
import functools

import jax
import jax.numpy as jnp
from jax import lax
from jax.experimental import pallas as pl
from jax.experimental.pallas import tpu as pltpu
from jax.experimental.pallas import tpu_sc as plsc

N = 10000
E = 320000
HEADS = 8
MID = 8
OUT = 64
IN = 128
SLOPE = 0.28

NC = 2
NS = 16
NW = NC * NS
L = 16

NP = 10112
EP = 327680
CA = 128
CB2 = 64
RPT = NP // NS

NORTH = 1
ETN = 10240
ETS = 10240
NT = NS * ETN

_f32 = jnp.float32

_GDN = lax.GatherDimensionNumbers(
    offset_dims=(), collapsed_slice_dims=(0,), start_index_map=(0,))


def _lgather(v, idx):
  return lax.gather(v, idx.reshape(L, 1), _GDN, (1,),
                    mode=lax.GatherScatterMode.PROMISE_IN_BOUNDS)


def _bcast(v, lane):
  return _lgather(v, jnp.full((L,), lane, jnp.int32))


def _sc_mesh():
  return plsc.VectorSubcoreMesh(
      core_axis_name="c", subcore_axis_name="s", num_cores=NC,
      num_subcores=NS)


def _passA_body(src_h, dst_h, as_h, ad_h, g_h, z16_h, ea_h, den_h,
                idxs, idxd, is0, id0, is1, id1, ids0, ids1,
                asb0, adb0, asb1, adb1, eab0, eab1, gbuf,
                gsem0, gsem1, wsem0, wsem1, den_sp):
  cid = lax.axis_index("c")
  sid = lax.axis_index("s")
  row0 = pl.multiple_of(sid * RPT, 8)
  for i in range(RPT // CA):
    pltpu.sync_copy(z16_h, den_sp.at[pl.ds(row0 + i * CA, CA)])
  pltpu.sync_copy(z16_h.at[pl.ds(0, RPT % CA)],
                  den_sp.at[pl.ds(row0 + (RPT // CA) * CA, RPT % CA)])
  pltpu.sync_copy(g_h, gbuf)
  plsc.subcore_barrier()
  gv = gbuf[...]

  def ldidx(src2d, k, dst1d):
    for j in range(CA // L):
      dst1d[pl.ds(j * L, L)] = src2d[k * (CA // L) + j]

  def pipe(core_base, ett):
    er = ett // L
    rb = pl.multiple_of((core_base + sid * ett) // L, 8)
    pltpu.sync_copy(src_h.at[pl.ds(rb, er)], idxs.at[pl.ds(0, er)])
    pltpu.sync_copy(dst_h.at[pl.ds(rb, er)], idxd.at[pl.ds(0, er)])
    base0 = core_base + sid * ett

    def issue(k, isb, idb_, asb, adb, sem):
      ldidx(idxs, k, isb)
      ldidx(idxd, k, idb_)
      pltpu.async_copy(as_h.at[isb], asb, sem)
      pltpu.async_copy(ad_h.at[idb_], adb, sem)

    def wait_g(isb, idb_, asb, adb, sem):
      pltpu.make_async_copy(as_h.at[isb], asb, sem).wait()
      pltpu.make_async_copy(ad_h.at[idb_], adb, sem).wait()

    def compute(k, sidb, asb, adb, eab, wsem):
      def edge(e, cc):
        v = asb[e] + adb[e]
        a = jnp.maximum(v, v * SLOPE) - gv
        eab[e] = jnp.exp(a)
        return cc
      lax.fori_loop(0, CA, edge, 0, unroll=4)
      base = pl.multiple_of(base0 + k * CA, CA)
      pltpu.async_copy(eab, ea_h.at[pl.ds(base, CA)], wsem)
      ldidx(idxd, k, sidb)
      pltpu.sync_copy(eab, den_sp.at[sidb], add=True)

    def wait_w(eab, wsem):
      pltpu.make_async_copy(eab, ea_h.at[pl.ds(0, CA)], wsem).wait()

    nh = ett // CA // 2
    issue(0, is0, id0, asb0, adb0, gsem0)

    def body(k2, cc):
      k = 2 * k2
      issue(k + 1, is1, id1, asb1, adb1, gsem1)
      wait_g(is0, id0, asb0, adb0, gsem0)

      @pl.when(k2 > 0)
      def _():
        wait_w(eab0, wsem0)
      compute(k, ids0, asb0, adb0, eab0, wsem0)

      @pl.when(k2 < nh - 1)
      def _():
        issue(k + 2, is0, id0, asb0, adb0, gsem0)
      wait_g(is1, id1, asb1, adb1, gsem1)

      @pl.when(k2 > 0)
      def _():
        wait_w(eab1, wsem1)
      compute(k + 1, ids1, asb1, adb1, eab1, wsem1)
      return cc

    lax.fori_loop(0, nh, body, 0)
    wait_w(eab0, wsem0)
    wait_w(eab1, wsem1)

  @pl.when(cid == NORTH)
  def _():
    pipe(0, ETN)

  @pl.when(cid != NORTH)
  def _():
    pipe(NT, ETS)

  plsc.subcore_barrier()
  pltpu.sync_copy(den_sp.at[pl.ds(row0, RPT)],
                  den_h.at[cid, pl.ds(row0, RPT)])


_passA = functools.partial(
    pl.kernel, _passA_body,
    out_type=[jax.ShapeDtypeStruct((EP, L), _f32),
              jax.ShapeDtypeStruct((NC, NP, L), _f32)],
    mesh=_sc_mesh(),
    compiler_params=pltpu.CompilerParams(use_tc_tiling_on_sc=False),
    scratch_types=[pltpu.VMEM((ETN // L, L), jnp.int32),
                   pltpu.VMEM((ETN // L, L), jnp.int32),
                   pltpu.VMEM((CA,), jnp.int32),
                   pltpu.VMEM((CA,), jnp.int32),
                   pltpu.VMEM((CA,), jnp.int32),
                   pltpu.VMEM((CA,), jnp.int32),
                   pltpu.VMEM((CA,), jnp.int32),
                   pltpu.VMEM((CA,), jnp.int32),
                   pltpu.VMEM((CA, L), _f32),
                   pltpu.VMEM((CA, L), _f32),
                   pltpu.VMEM((CA, L), _f32),
                   pltpu.VMEM((CA, L), _f32),
                   pltpu.VMEM((CA, L), _f32),
                   pltpu.VMEM((CA, L), _f32),
                   pltpu.VMEM((L,), _f32),
                   pltpu.SemaphoreType.DMA,
                   pltpu.SemaphoreType.DMA,
                   pltpu.SemaphoreType.DMA,
                   pltpu.SemaphoreType.DMA,
                   pltpu.VMEM_SHARED((NP, L), _f32)])


def _passB1_body(src_h, dst_h, ea_h, inv_h, htab_h, z4_h, o_h,
                 idxs, idxd, is0, id0, is1, id1, ids0, ids1,
                 hb0, idb0, eab0, hb1, idb1, eab1, msg0, msg1,
                 gsem0, gsem1, ssem0, ssem1, out_sp):
  cid = lax.axis_index("c")
  sid = lax.axis_index("s")
  row0 = pl.multiple_of(sid * RPT, 8)
  for i in range(RPT // CA):
    pltpu.sync_copy(z4_h, out_sp.at[pl.ds(row0 + i * CA, CA)])
  pltpu.sync_copy(z4_h.at[pl.ds(0, RPT % CA)],
                  out_sp.at[pl.ds(row0 + (RPT // CA) * CA, RPT % CA)])
  plsc.subcore_barrier()
  half = lax.shift_right_logical(
      lax.broadcasted_iota(jnp.int32, (L,), 0), 3)

  def ldidx(src2d, k, dst1d):
    for j in range(CA // L):
      dst1d[pl.ds(j * L, L)] = src2d[k * (CA // L) + j]

  def pipe(core_base, ett):
    er = ett // L
    rb = pl.multiple_of((core_base + sid * ett) // L, 8)
    pltpu.sync_copy(src_h.at[pl.ds(rb, er)], idxs.at[pl.ds(0, er)])
    pltpu.sync_copy(dst_h.at[pl.ds(rb, er)], idxd.at[pl.ds(0, er)])
    base0 = core_base + sid * ett

    def issue(k, isb, idb_, hb, idb, eab, sem):
      ldidx(idxs, k, isb)
      ldidx(idxd, k, idb_)
      pltpu.async_copy(htab_h.at[isb], hb, sem)
      pltpu.async_copy(inv_h.at[idb_], idb, sem)
      base = pl.multiple_of(base0 + k * CA, CA)
      pltpu.async_copy(ea_h.at[pl.ds(base, CA)], eab, sem)

    def wait_g(isb, idb_, hb, idb, eab, sem):
      pltpu.make_async_copy(htab_h.at[isb], hb, sem).wait()
      pltpu.make_async_copy(inv_h.at[idb_], idb, sem).wait()
      pltpu.make_async_copy(ea_h.at[pl.ds(0, CA)], eab, sem).wait()

    def compute(k, sidb, hb, idb, eab, msg, ssem):
      ldidx(idxd, k, sidb)

      def edge(e, cc):
        cv = eab[e] * idb[e]
        for j in range(4):
          cj = _lgather(cv, 2 * j + half)
          msg[e, j] = hb[e, j] * cj
        return cc
      lax.fori_loop(0, CA, edge, 0, unroll=4)
      pltpu.async_copy(msg, out_sp.at[sidb], ssem, add=True)

    def wait_s(sidb, msg, ssem):
      pltpu.make_async_copy(msg, out_sp.at[sidb], ssem).wait()

    nh = ett // CA // 2
    issue(0, is0, id0, hb0, idb0, eab0, gsem0)

    def body(k2, cc):
      k = 2 * k2
      issue(k + 1, is1, id1, hb1, idb1, eab1, gsem1)
      wait_g(is0, id0, hb0, idb0, eab0, gsem0)

      @pl.when(k2 > 0)
      def _():
        wait_s(ids0, msg0, ssem0)
      compute(k, ids0, hb0, idb0, eab0, msg0, ssem0)

      @pl.when(k2 < nh - 1)
      def _():
        issue(k + 2, is0, id0, hb0, idb0, eab0, gsem0)
      wait_g(is1, id1, hb1, idb1, eab1, gsem1)

      @pl.when(k2 > 0)
      def _():
        wait_s(ids1, msg1, ssem1)
      compute(k + 1, ids1, hb1, idb1, eab1, msg1, ssem1)
      return cc

    lax.fori_loop(0, nh, body, 0)
    wait_s(ids0, msg0, ssem0)
    wait_s(ids1, msg1, ssem1)

  @pl.when(cid == NORTH)
  def _():
    pipe(0, ETN)

  @pl.when(cid != NORTH)
  def _():
    pipe(NT, ETS)

  plsc.subcore_barrier()
  pltpu.sync_copy(out_sp.at[pl.ds(row0, RPT)],
                  o_h.at[cid, pl.ds(row0, RPT)])


_passB1 = functools.partial(
    pl.kernel, _passB1_body,
    out_type=[jax.ShapeDtypeStruct((NC, NP, 4, L), _f32)],
    mesh=_sc_mesh(),
    compiler_params=pltpu.CompilerParams(use_tc_tiling_on_sc=False),
    scratch_types=[pltpu.VMEM((ETN // L, L), jnp.int32),
                   pltpu.VMEM((ETN // L, L), jnp.int32),
                   pltpu.VMEM((CA,), jnp.int32),
                   pltpu.VMEM((CA,), jnp.int32),
                   pltpu.VMEM((CA,), jnp.int32),
                   pltpu.VMEM((CA,), jnp.int32),
                   pltpu.VMEM((CA,), jnp.int32),
                   pltpu.VMEM((CA,), jnp.int32),
                   pltpu.VMEM((CA, 4, L), _f32),
                   pltpu.VMEM((CA, L), _f32),
                   pltpu.VMEM((CA, L), _f32),
                   pltpu.VMEM((CA, 4, L), _f32),
                   pltpu.VMEM((CA, L), _f32),
                   pltpu.VMEM((CA, L), _f32),
                   pltpu.VMEM((CA, 4, L), _f32),
                   pltpu.VMEM((CA, 4, L), _f32),
                   pltpu.SemaphoreType.DMA,
                   pltpu.SemaphoreType.DMA,
                   pltpu.SemaphoreType.DMA,
                   pltpu.SemaphoreType.DMA,
                   pltpu.VMEM_SHARED((NP, 4, L), _f32)])


def _passB2_body(src_h, dst_h, ea_h, inv_h, htab_h, z4_h, o_h,
                 is0, id0, is1, id1, ids0, ids1,
                 hb0, idb0, eab0, hb1, idb1, eab1, msg0, msg1,
                 gsem0, gsem1, ssem0, ssem1, isem0, isem1, out_sp):
  cid = lax.axis_index("c")
  sid = lax.axis_index("s")
  row0 = pl.multiple_of(sid * RPT, 8)
  for i in range(RPT // CA):
    pltpu.sync_copy(z4_h, out_sp.at[pl.ds(row0 + i * CA, CA)])
  pltpu.sync_copy(z4_h.at[pl.ds(0, RPT % CA)],
                  out_sp.at[pl.ds(row0 + (RPT // CA) * CA, RPT % CA)])
  plsc.subcore_barrier()

  def cpidx(idb_, sidb):
    for j in range(CB2 // L):
      sidb[pl.ds(j * L, L)] = idb_[pl.ds(j * L, L)]

  def pipe(core_base, ett):
    base0 = core_base + sid * ett

    def issue_idx(k, isb, idb_, isem):
      base = pl.multiple_of(base0 + k * CB2, CB2)
      pltpu.async_copy(src_h.at[pl.ds(base, CB2)], isb, isem)
      pltpu.async_copy(dst_h.at[pl.ds(base, CB2)], idb_, isem)

    def wait_idx(isb, idb_, isem):
      pltpu.make_async_copy(src_h.at[pl.ds(0, CB2)], isb, isem).wait()
      pltpu.make_async_copy(dst_h.at[pl.ds(0, CB2)], idb_, isem).wait()

    def issue_g(k, isb, idb_, hb, idb, eab, sem):
      pltpu.async_copy(htab_h.at[isb], hb, sem)
      pltpu.async_copy(inv_h.at[idb_], idb, sem)
      base = pl.multiple_of(base0 + k * CB2, CB2)
      pltpu.async_copy(ea_h.at[pl.ds(base, CB2)], eab, sem)

    def wait_g(isb, idb_, hb, idb, eab, sem):
      pltpu.make_async_copy(htab_h.at[isb], hb, sem).wait()
      pltpu.make_async_copy(inv_h.at[idb_], idb, sem).wait()
      pltpu.make_async_copy(ea_h.at[pl.ds(0, CB2)], eab, sem).wait()

    def compute(hb, idb, eab, msg, sidb, ssem):
      def edge(e, cc):
        cv = eab[e] * idb[e]
        acc = [None] * 4
        for h in range(HEADS):
          cf = _bcast(cv, h)
          for p in range(2):
            w = hb[e, 2 * h + p]
            lo = lax.bitcast_convert_type(
                lax.shift_left(w, jnp.int32(16)), _f32)
            hi = lax.bitcast_convert_type(
                lax.bitwise_and(w, jnp.int32(-65536)), _f32)
            for q, g in ((0, lo), (1, hi)):
              j = 2 * p + q
              t = g * cf
              acc[j] = t if acc[j] is None else acc[j] + t
        for j in range(4):
          msg[e, j] = acc[j]
        return cc
      lax.fori_loop(0, CB2, edge, 0, unroll=2)
      pltpu.async_copy(msg, out_sp.at[sidb], ssem, add=True)

    def wait_s(sidb, msg, ssem):
      pltpu.make_async_copy(msg, out_sp.at[sidb], ssem).wait()

    nh = ett // CB2 // 2
    issue_idx(0, is0, id0, isem0)
    issue_idx(1, is1, id1, isem1)
    wait_idx(is0, id0, isem0)
    issue_g(0, is0, id0, hb0, idb0, eab0, gsem0)

    def body(k2, cc):
      k = 2 * k2
      wait_idx(is1, id1, isem1)
      issue_g(k + 1, is1, id1, hb1, idb1, eab1, gsem1)

      wait_g(is0, id0, hb0, idb0, eab0, gsem0)

      @pl.when(k2 > 0)
      def _():
        wait_s(ids0, msg0, ssem0)
      cpidx(id0, ids0)

      @pl.when(k2 < nh - 1)
      def _():
        issue_idx(k + 2, is0, id0, isem0)
      compute(hb0, idb0, eab0, msg0, ids0, ssem0)

      @pl.when(k2 < nh - 1)
      def _():
        wait_idx(is0, id0, isem0)
        issue_g(k + 2, is0, id0, hb0, idb0, eab0, gsem0)

      wait_g(is1, id1, hb1, idb1, eab1, gsem1)

      @pl.when(k2 > 0)
      def _():
        wait_s(ids1, msg1, ssem1)
      cpidx(id1, ids1)

      @pl.when(k2 < nh - 1)
      def _():
        issue_idx(k + 3, is1, id1, isem1)
      compute(hb1, idb1, eab1, msg1, ids1, ssem1)
      return cc

    lax.fori_loop(0, nh, body, 0)
    wait_s(ids0, msg0, ssem0)
    wait_s(ids1, msg1, ssem1)

  @pl.when(cid == NORTH)
  def _():
    pipe(0, ETN)

  @pl.when(cid != NORTH)
  def _():
    pipe(NT, ETS)

  plsc.subcore_barrier()
  pltpu.sync_copy(out_sp.at[pl.ds(row0, RPT)],
                  o_h.at[cid, pl.ds(row0, RPT)])


_passB2 = functools.partial(
    pl.kernel, _passB2_body,
    out_type=[jax.ShapeDtypeStruct((NC, NP, 4, L), _f32)],
    mesh=_sc_mesh(),
    compiler_params=pltpu.CompilerParams(use_tc_tiling_on_sc=False),
    scratch_types=[pltpu.VMEM((CB2,), jnp.int32),
                   pltpu.VMEM((CB2,), jnp.int32),
                   pltpu.VMEM((CB2,), jnp.int32),
                   pltpu.VMEM((CB2,), jnp.int32),
                   pltpu.VMEM((CB2,), jnp.int32),
                   pltpu.VMEM((CB2,), jnp.int32),
                   pltpu.VMEM((CB2, 16, L), jnp.int32),
                   pltpu.VMEM((CB2, L), _f32),
                   pltpu.VMEM((CB2, L), _f32),
                   pltpu.VMEM((CB2, 16, L), jnp.int32),
                   pltpu.VMEM((CB2, L), _f32),
                   pltpu.VMEM((CB2, L), _f32),
                   pltpu.VMEM((CB2, 4, L), _f32),
                   pltpu.VMEM((CB2, 4, L), _f32),
                   pltpu.SemaphoreType.DMA,
                   pltpu.SemaphoreType.DMA,
                   pltpu.SemaphoreType.DMA,
                   pltpu.SemaphoreType.DMA,
                   pltpu.SemaphoreType.DMA,
                   pltpu.SemaphoreType.DMA,
                   pltpu.VMEM_SHARED((NP, 4, L), _f32)])


def _tc1_body(xp_ref, w_ref, h_ref, as_ref, ad_ref, g_ref):
  hcat = jnp.dot(xp_ref[...], w_ref[...], preferred_element_type=_f32)
  h_ref[...] = hcat[:, :64]
  a_s = hcat[:, 64:80]
  a_d = hcat[:, 80:96]
  as_ref[...] = a_s
  ad_ref[...] = a_d
  g_ref[...] = (jnp.max(a_s, axis=0) + jnp.max(a_d, axis=0)).reshape(1, L)


def _dsum_body(scale, den_ref, inv_ref):
  d = den_ref[0] + den_ref[1]
  inv_ref[...] = scale / (d + 1e-16)


def _tc2_body(o_ref, b1_ref, w2_ref, as2_ref, ad2_ref,
              h2_ref, a2s_ref, a2d_ref, g2_ref):
  z = o_ref[0] + o_ref[1] + b1_ref[...]
  z = jnp.where(z > 0, z, jnp.exp(jnp.minimum(z, 0.0)) - 1.0)
  ri = lax.broadcasted_iota(jnp.int32, (NP, 64), 0)
  z = jnp.where(ri < N, z, 0.0)
  h2 = jnp.dot(z, w2_ref[...], preferred_element_type=_f32)
  h2_ref[...] = h2
  a2s = jnp.dot(h2, as2_ref[...], preferred_element_type=_f32)
  a2d = jnp.dot(h2, ad2_ref[...], preferred_element_type=_f32)
  a2s_ref[...] = a2s
  a2d_ref[...] = a2d
  g2_ref[...] = (jnp.max(a2s, axis=0) + jnp.max(a2d, axis=0)).reshape(1, L)


def _tc3_body(o_ref, b2_ref, out_ref):
  o = o_ref[0] + o_ref[1] + b2_ref[...]
  ri = lax.broadcasted_iota(jnp.int32, (NP, OUT), 0)
  o = jnp.where(ri < N, o, 0.0)
  m = jnp.max(o, axis=1, keepdims=True)
  o = o - m
  out_ref[...] = o - jnp.log(jnp.sum(jnp.exp(o), axis=1, keepdims=True))


def kernel(node_feature, adj_list, W1, att_src1, att_dst1, b1,
           W2, att_src2, att_dst2, b2):
  W1r = W1.reshape(IN, HEADS, MID)
  AS1 = jnp.pad(jnp.einsum("ihm,hm->ih", W1r, att_src1),
                ((0, 0), (0, L - HEADS)))
  AD1 = jnp.pad(jnp.einsum("ihm,hm->ih", W1r, att_dst1),
                ((0, 0), (0, L - HEADS)))
  Wcat = jnp.concatenate([W1, AS1, AD1], axis=1)

  eye8 = jnp.eye(HEADS, dtype=_f32)
  AS2 = jnp.pad((att_src2[:, :, None] * eye8[:, None, :])
                .reshape(HEADS * OUT, HEADS), ((0, 0), (0, L - HEADS)))
  AD2 = jnp.pad((att_dst2[:, :, None] * eye8[:, None, :])
                .reshape(HEADS * OUT, HEADS), ((0, 0), (0, L - HEADS)))

  xp = jnp.pad(node_feature, ((0, NP - N), (0, 0)))
  pad_e = jnp.full((EP - E,), N, jnp.int32)
  srcp = jnp.concatenate([adj_list[0].astype(jnp.int32), pad_e])
  dstp = jnp.concatenate([adj_list[1].astype(jnp.int32), pad_e])
  srcA = srcp.reshape(EP // L, L)
  dstA = dstp.reshape(EP // L, L)

  z16 = jnp.zeros((CA, L), _f32)
  z4 = jnp.zeros((CA, 4, L), _f32)

  h1t, a1s, a1d, g1 = pl.pallas_call(
      _tc1_body,
      out_shape=[jax.ShapeDtypeStruct((NP, 64), _f32),
                 jax.ShapeDtypeStruct((NP, L), _f32),
                 jax.ShapeDtypeStruct((NP, L), _f32),
                 jax.ShapeDtypeStruct((1, L), _f32)])(xp, Wcat)

  ea1, den1 = _passA()(srcA, dstA, a1s, a1d, g1.reshape(L), z16)
  inv1 = pl.pallas_call(
      functools.partial(_dsum_body, 1.0),
      out_shape=jax.ShapeDtypeStruct((NP, L), _f32))(den1)
  (o1,) = _passB1()(srcA, dstA, ea1, inv1, h1t.reshape(NP, 4, L), z4)

  h2t, a2s, a2d, g2 = pl.pallas_call(
      _tc2_body,
      out_shape=[jax.ShapeDtypeStruct((NP, HEADS * OUT), _f32),
                 jax.ShapeDtypeStruct((NP, L), _f32),
                 jax.ShapeDtypeStruct((NP, L), _f32),
                 jax.ShapeDtypeStruct((1, L), _f32)])(
                     o1.reshape(NC, NP, 64), b1.reshape(1, 64), W2, AS2, AD2)

  ea2, den2 = _passA()(srcA, dstA, a2s, a2d, g2.reshape(L), z16)
  inv2 = pl.pallas_call(
      functools.partial(_dsum_body, 1.0 / HEADS),
      out_shape=jax.ShapeDtypeStruct((NP, L), _f32))(den2)
  h2q = lax.bitcast_convert_type(
      h2t.reshape(NP, L, 2, L).swapaxes(2, 3).astype(jnp.bfloat16),
      jnp.int32)
  (o2,) = _passB2()(srcp, dstp, ea2, inv2, h2q.reshape(NP, L, L), z4)

  outp = pl.pallas_call(
      _tc3_body,
      out_shape=jax.ShapeDtypeStruct((NP, OUT), _f32))(
          o2.reshape(NC, NP, OUT), b2.reshape(1, OUT))
  return outp[:N]


# speedup vs baseline: 1.1401x; 1.0637x over previous
import functools

import jax
import jax.numpy as jnp
from jax import lax
from jax.experimental import pallas as pl
from jax.experimental.pallas import tpu as pltpu
from jax.experimental.pallas import tpu_sc as plsc

N = 10000
E = 320000
HEADS = 8
MID = 8
OUT = 64
IN = 128
SLOPE = 0.28

NC = 2
NS = 16
NW = NC * NS
L = 16

NP = 10112
EP = 327680
CA = 128
CB2 = 64
RPT = NP // NS

NORTH = 1
ETN = 10240
ETS = 10240
NT = NS * ETN

_f32 = jnp.float32

_GDN = lax.GatherDimensionNumbers(
    offset_dims=(), collapsed_slice_dims=(0,), start_index_map=(0,))


def _lgather(v, idx):
  return lax.gather(v, idx.reshape(L, 1), _GDN, (1,),
                    mode=lax.GatherScatterMode.PROMISE_IN_BOUNDS)


def _bcast(v, lane):
  return _lgather(v, jnp.full((L,), lane, jnp.int32))


def _sc_mesh():
  return plsc.VectorSubcoreMesh(
      core_axis_name="c", subcore_axis_name="s", num_cores=NC,
      num_subcores=NS)


def _passA_body(src_h, dst_h, as_h, ad_h, g_h, z16_h, ea_h, den_h,
                idxs, idxd, is0, id0, is1, id1, ids0, ids1,
                asb0, adb0, asb1, adb1, eab0, eab1, gbuf,
                gsem0, gsem1, wsem0, wsem1, den_sp):
  cid = lax.axis_index("c")
  sid = lax.axis_index("s")
  row0 = pl.multiple_of(sid * RPT, 8)
  for i in range(RPT // CA):
    pltpu.sync_copy(z16_h, den_sp.at[pl.ds(row0 + i * CA, CA)])
  pltpu.sync_copy(z16_h.at[pl.ds(0, RPT % CA)],
                  den_sp.at[pl.ds(row0 + (RPT // CA) * CA, RPT % CA)])
  pltpu.sync_copy(g_h, gbuf)
  plsc.subcore_barrier()
  gv = gbuf[...]

  def ldidx(src2d, k, dst1d):
    for j in range(CA // L):
      dst1d[pl.ds(j * L, L)] = src2d[k * (CA // L) + j]

  def pipe(core_base, ett):
    er = ett // L
    rb = pl.multiple_of((core_base + sid * ett) // L, 8)
    pltpu.sync_copy(src_h.at[pl.ds(rb, er)], idxs.at[pl.ds(0, er)])
    pltpu.sync_copy(dst_h.at[pl.ds(rb, er)], idxd.at[pl.ds(0, er)])
    base0 = core_base + sid * ett

    def issue(k, isb, idb_, asb, adb, sem):
      ldidx(idxs, k, isb)
      ldidx(idxd, k, idb_)
      pltpu.async_copy(as_h.at[isb], asb, sem)
      pltpu.async_copy(ad_h.at[idb_], adb, sem)

    def wait_g(isb, idb_, asb, adb, sem):
      pltpu.make_async_copy(as_h.at[isb], asb, sem).wait()
      pltpu.make_async_copy(ad_h.at[idb_], adb, sem).wait()

    def compute(k, sidb, asb, adb, eab, wsem):
      def edge(e, cc):
        v = asb[e] + adb[e]
        a = jnp.maximum(v, v * SLOPE) - gv
        eab[e] = jnp.exp(a)
        return cc
      lax.fori_loop(0, CA, edge, 0, unroll=4)
      base = pl.multiple_of(base0 + k * CA, CA)
      pltpu.async_copy(eab, ea_h.at[pl.ds(base, CA)], wsem)
      ldidx(idxd, k, sidb)
      pltpu.sync_copy(eab, den_sp.at[sidb], add=True)

    def wait_w(eab, wsem):
      pltpu.make_async_copy(eab, ea_h.at[pl.ds(0, CA)], wsem).wait()

    nh = ett // CA // 2
    issue(0, is0, id0, asb0, adb0, gsem0)

    def body(k2, cc):
      k = 2 * k2
      issue(k + 1, is1, id1, asb1, adb1, gsem1)
      wait_g(is0, id0, asb0, adb0, gsem0)

      @pl.when(k2 > 0)
      def _():
        wait_w(eab0, wsem0)
      compute(k, ids0, asb0, adb0, eab0, wsem0)

      @pl.when(k2 < nh - 1)
      def _():
        issue(k + 2, is0, id0, asb0, adb0, gsem0)
      wait_g(is1, id1, asb1, adb1, gsem1)

      @pl.when(k2 > 0)
      def _():
        wait_w(eab1, wsem1)
      compute(k + 1, ids1, asb1, adb1, eab1, wsem1)
      return cc

    lax.fori_loop(0, nh, body, 0)
    wait_w(eab0, wsem0)
    wait_w(eab1, wsem1)

  @pl.when(cid == NORTH)
  def _():
    pipe(0, ETN)

  @pl.when(cid != NORTH)
  def _():
    pipe(NT, ETS)

  plsc.subcore_barrier()
  pltpu.sync_copy(den_sp.at[pl.ds(row0, RPT)],
                  den_h.at[cid, pl.ds(row0, RPT)])


_passA = functools.partial(
    pl.kernel, _passA_body,
    out_type=[jax.ShapeDtypeStruct((EP, L), _f32),
              jax.ShapeDtypeStruct((NC, NP, L), _f32)],
    mesh=_sc_mesh(),
    compiler_params=pltpu.CompilerParams(use_tc_tiling_on_sc=False),
    scratch_types=[pltpu.VMEM((ETN // L, L), jnp.int32),
                   pltpu.VMEM((ETN // L, L), jnp.int32),
                   pltpu.VMEM((CA,), jnp.int32),
                   pltpu.VMEM((CA,), jnp.int32),
                   pltpu.VMEM((CA,), jnp.int32),
                   pltpu.VMEM((CA,), jnp.int32),
                   pltpu.VMEM((CA,), jnp.int32),
                   pltpu.VMEM((CA,), jnp.int32),
                   pltpu.VMEM((CA, L), _f32),
                   pltpu.VMEM((CA, L), _f32),
                   pltpu.VMEM((CA, L), _f32),
                   pltpu.VMEM((CA, L), _f32),
                   pltpu.VMEM((CA, L), _f32),
                   pltpu.VMEM((CA, L), _f32),
                   pltpu.VMEM((L,), _f32),
                   pltpu.SemaphoreType.DMA,
                   pltpu.SemaphoreType.DMA,
                   pltpu.SemaphoreType.DMA,
                   pltpu.SemaphoreType.DMA,
                   pltpu.VMEM_SHARED((NP, L), _f32)])


def _passB1_body(src_h, dst_h, ea_h, htab_h, z4_h, o_h,
                 idxs, idxd, is0, id0, is1, id1, ids0, ids1,
                 hb0, eab0, hb1, eab1, msg0, msg1,
                 gsem0, gsem1, ssem0, ssem1, out_sp):
  cid = lax.axis_index("c")
  sid = lax.axis_index("s")
  row0 = pl.multiple_of(sid * RPT, 8)
  for i in range(RPT // CA):
    pltpu.sync_copy(z4_h, out_sp.at[pl.ds(row0 + i * CA, CA)])
  pltpu.sync_copy(z4_h.at[pl.ds(0, RPT % CA)],
                  out_sp.at[pl.ds(row0 + (RPT // CA) * CA, RPT % CA)])
  plsc.subcore_barrier()
  half = lax.shift_right_logical(
      lax.broadcasted_iota(jnp.int32, (L,), 0), 3)

  def ldidx(src2d, k, dst1d):
    for j in range(CA // L):
      dst1d[pl.ds(j * L, L)] = src2d[k * (CA // L) + j]

  def pipe(core_base, ett):
    er = ett // L
    rb = pl.multiple_of((core_base + sid * ett) // L, 8)
    pltpu.sync_copy(src_h.at[pl.ds(rb, er)], idxs.at[pl.ds(0, er)])
    pltpu.sync_copy(dst_h.at[pl.ds(rb, er)], idxd.at[pl.ds(0, er)])
    base0 = core_base + sid * ett

    def issue(k, isb, hb, eab, sem):
      ldidx(idxs, k, isb)
      pltpu.async_copy(htab_h.at[isb], hb, sem)
      base = pl.multiple_of(base0 + k * CA, CA)
      pltpu.async_copy(ea_h.at[pl.ds(base, CA)], eab, sem)

    def wait_g(isb, hb, eab, sem):
      pltpu.make_async_copy(htab_h.at[isb], hb, sem).wait()
      pltpu.make_async_copy(ea_h.at[pl.ds(0, CA)], eab, sem).wait()

    def compute(k, sidb, hb, eab, msg, ssem):
      ldidx(idxd, k, sidb)

      def edge(e, cc):
        cv = eab[e]
        for j in range(4):
          cj = _lgather(cv, 2 * j + half)
          msg[e, j] = hb[e, j] * cj
        return cc
      lax.fori_loop(0, CA, edge, 0, unroll=4)
      pltpu.async_copy(msg, out_sp.at[sidb], ssem, add=True)

    def wait_s(sidb, msg, ssem):
      pltpu.make_async_copy(msg, out_sp.at[sidb], ssem).wait()

    nh = ett // CA // 2
    issue(0, is0, hb0, eab0, gsem0)

    def body(k2, cc):
      k = 2 * k2
      issue(k + 1, is1, hb1, eab1, gsem1)
      wait_g(is0, hb0, eab0, gsem0)

      @pl.when(k2 > 0)
      def _():
        wait_s(ids0, msg0, ssem0)
      compute(k, ids0, hb0, eab0, msg0, ssem0)

      @pl.when(k2 < nh - 1)
      def _():
        issue(k + 2, is0, hb0, eab0, gsem0)
      wait_g(is1, hb1, eab1, gsem1)

      @pl.when(k2 > 0)
      def _():
        wait_s(ids1, msg1, ssem1)
      compute(k + 1, ids1, hb1, eab1, msg1, ssem1)
      return cc

    lax.fori_loop(0, nh, body, 0)
    wait_s(ids0, msg0, ssem0)
    wait_s(ids1, msg1, ssem1)

  @pl.when(cid == NORTH)
  def _():
    pipe(0, ETN)

  @pl.when(cid != NORTH)
  def _():
    pipe(NT, ETS)

  plsc.subcore_barrier()
  pltpu.sync_copy(out_sp.at[pl.ds(row0, RPT)],
                  o_h.at[cid, pl.ds(row0, RPT)])


_passB1 = functools.partial(
    pl.kernel, _passB1_body,
    out_type=[jax.ShapeDtypeStruct((NC, NP, 4, L), _f32)],
    mesh=_sc_mesh(),
    compiler_params=pltpu.CompilerParams(use_tc_tiling_on_sc=False),
    scratch_types=[pltpu.VMEM((ETN // L, L), jnp.int32),
                   pltpu.VMEM((ETN // L, L), jnp.int32),
                   pltpu.VMEM((CA,), jnp.int32),
                   pltpu.VMEM((CA,), jnp.int32),
                   pltpu.VMEM((CA,), jnp.int32),
                   pltpu.VMEM((CA,), jnp.int32),
                   pltpu.VMEM((CA,), jnp.int32),
                   pltpu.VMEM((CA,), jnp.int32),
                   pltpu.VMEM((CA, 4, L), _f32),
                   pltpu.VMEM((CA, L), _f32),
                   pltpu.VMEM((CA, 4, L), _f32),
                   pltpu.VMEM((CA, L), _f32),
                   pltpu.VMEM((CA, 4, L), _f32),
                   pltpu.VMEM((CA, 4, L), _f32),
                   pltpu.SemaphoreType.DMA,
                   pltpu.SemaphoreType.DMA,
                   pltpu.SemaphoreType.DMA,
                   pltpu.SemaphoreType.DMA,
                   pltpu.VMEM_SHARED((NP, 4, L), _f32)])


def _passB2_body(src_h, dst_h, ea_h, inv_h, htab_h, z4_h, o_h,
                 is0, id0, is1, id1, ids0, ids1,
                 hb0, idb0, eab0, hb1, idb1, eab1, msg0, msg1,
                 gsem0, gsem1, ssem0, ssem1, isem0, isem1, out_sp):
  cid = lax.axis_index("c")
  sid = lax.axis_index("s")
  row0 = pl.multiple_of(sid * RPT, 8)
  for i in range(RPT // CA):
    pltpu.sync_copy(z4_h, out_sp.at[pl.ds(row0 + i * CA, CA)])
  pltpu.sync_copy(z4_h.at[pl.ds(0, RPT % CA)],
                  out_sp.at[pl.ds(row0 + (RPT // CA) * CA, RPT % CA)])
  plsc.subcore_barrier()

  def cpidx(idb_, sidb):
    for j in range(CB2 // L):
      sidb[pl.ds(j * L, L)] = idb_[pl.ds(j * L, L)]

  def pipe(core_base, ett):
    base0 = core_base + sid * ett

    def issue_idx(k, isb, idb_, isem):
      base = pl.multiple_of(base0 + k * CB2, CB2)
      pltpu.async_copy(src_h.at[pl.ds(base, CB2)], isb, isem)
      pltpu.async_copy(dst_h.at[pl.ds(base, CB2)], idb_, isem)

    def wait_idx(isb, idb_, isem):
      pltpu.make_async_copy(src_h.at[pl.ds(0, CB2)], isb, isem).wait()
      pltpu.make_async_copy(dst_h.at[pl.ds(0, CB2)], idb_, isem).wait()

    def issue_g(k, isb, idb_, hb, idb, eab, sem):
      pltpu.async_copy(htab_h.at[isb], hb, sem)
      pltpu.async_copy(inv_h.at[idb_], idb, sem)
      base = pl.multiple_of(base0 + k * CB2, CB2)
      pltpu.async_copy(ea_h.at[pl.ds(base, CB2)], eab, sem)

    def wait_g(isb, idb_, hb, idb, eab, sem):
      pltpu.make_async_copy(htab_h.at[isb], hb, sem).wait()
      pltpu.make_async_copy(inv_h.at[idb_], idb, sem).wait()
      pltpu.make_async_copy(ea_h.at[pl.ds(0, CB2)], eab, sem).wait()

    def compute(hb, idb, eab, msg, sidb, ssem):
      def edge(e, cc):
        cv = eab[e] * idb[e]
        acc = [None] * 4
        for h in range(HEADS):
          cf = _bcast(cv, h)
          for p in range(2):
            w = hb[e, 2 * h + p]
            lo = lax.bitcast_convert_type(
                lax.shift_left(w, jnp.int32(16)), _f32)
            hi = lax.bitcast_convert_type(
                lax.bitwise_and(w, jnp.int32(-65536)), _f32)
            for q, g in ((0, lo), (1, hi)):
              j = 2 * p + q
              t = g * cf
              acc[j] = t if acc[j] is None else acc[j] + t
        for j in range(4):
          msg[e, j] = acc[j]
        return cc
      lax.fori_loop(0, CB2, edge, 0, unroll=2)
      pltpu.async_copy(msg, out_sp.at[sidb], ssem, add=True)

    def wait_s(sidb, msg, ssem):
      pltpu.make_async_copy(msg, out_sp.at[sidb], ssem).wait()

    nh = ett // CB2 // 2
    issue_idx(0, is0, id0, isem0)
    issue_idx(1, is1, id1, isem1)
    wait_idx(is0, id0, isem0)
    issue_g(0, is0, id0, hb0, idb0, eab0, gsem0)

    def body(k2, cc):
      k = 2 * k2
      wait_idx(is1, id1, isem1)
      issue_g(k + 1, is1, id1, hb1, idb1, eab1, gsem1)

      wait_g(is0, id0, hb0, idb0, eab0, gsem0)

      @pl.when(k2 > 0)
      def _():
        wait_s(ids0, msg0, ssem0)
      cpidx(id0, ids0)

      @pl.when(k2 < nh - 1)
      def _():
        issue_idx(k + 2, is0, id0, isem0)
      compute(hb0, idb0, eab0, msg0, ids0, ssem0)

      @pl.when(k2 < nh - 1)
      def _():
        wait_idx(is0, id0, isem0)
        issue_g(k + 2, is0, id0, hb0, idb0, eab0, gsem0)

      wait_g(is1, id1, hb1, idb1, eab1, gsem1)

      @pl.when(k2 > 0)
      def _():
        wait_s(ids1, msg1, ssem1)
      cpidx(id1, ids1)

      @pl.when(k2 < nh - 1)
      def _():
        issue_idx(k + 3, is1, id1, isem1)
      compute(hb1, idb1, eab1, msg1, ids1, ssem1)
      return cc

    lax.fori_loop(0, nh, body, 0)
    wait_s(ids0, msg0, ssem0)
    wait_s(ids1, msg1, ssem1)

  @pl.when(cid == NORTH)
  def _():
    pipe(0, ETN)

  @pl.when(cid != NORTH)
  def _():
    pipe(NT, ETS)

  plsc.subcore_barrier()
  pltpu.sync_copy(out_sp.at[pl.ds(row0, RPT)],
                  o_h.at[cid, pl.ds(row0, RPT)])


_passB2 = functools.partial(
    pl.kernel, _passB2_body,
    out_type=[jax.ShapeDtypeStruct((NC, NP, 4, L), _f32)],
    mesh=_sc_mesh(),
    compiler_params=pltpu.CompilerParams(use_tc_tiling_on_sc=False),
    scratch_types=[pltpu.VMEM((CB2,), jnp.int32),
                   pltpu.VMEM((CB2,), jnp.int32),
                   pltpu.VMEM((CB2,), jnp.int32),
                   pltpu.VMEM((CB2,), jnp.int32),
                   pltpu.VMEM((CB2,), jnp.int32),
                   pltpu.VMEM((CB2,), jnp.int32),
                   pltpu.VMEM((CB2, 16, L), jnp.int32),
                   pltpu.VMEM((CB2, L), _f32),
                   pltpu.VMEM((CB2, L), _f32),
                   pltpu.VMEM((CB2, 16, L), jnp.int32),
                   pltpu.VMEM((CB2, L), _f32),
                   pltpu.VMEM((CB2, L), _f32),
                   pltpu.VMEM((CB2, 4, L), _f32),
                   pltpu.VMEM((CB2, 4, L), _f32),
                   pltpu.SemaphoreType.DMA,
                   pltpu.SemaphoreType.DMA,
                   pltpu.SemaphoreType.DMA,
                   pltpu.SemaphoreType.DMA,
                   pltpu.SemaphoreType.DMA,
                   pltpu.SemaphoreType.DMA,
                   pltpu.VMEM_SHARED((NP, 4, L), _f32)])


def _tc1_body(xp_ref, w_ref, h_ref, as_ref, ad_ref, g_ref):
  hcat = jnp.dot(xp_ref[...], w_ref[...], preferred_element_type=_f32)
  h_ref[...] = hcat[:, :64]
  a_s = hcat[:, 64:80]
  a_d = hcat[:, 80:96]
  as_ref[...] = a_s
  ad_ref[...] = a_d
  g_ref[...] = (jnp.max(a_s, axis=0) + jnp.max(a_d, axis=0)).reshape(1, L)


def _dsum_body(scale, den_ref, inv_ref):
  d = den_ref[0] + den_ref[1]
  inv_ref[...] = scale / (d + 1e-16)


def _tc2_body(o_ref, inv_ref, r_ref, b1_ref, w2_ref, as2_ref, ad2_ref,
              h2_ref, a2s_ref, a2d_ref, g2_ref):
  inv64 = jnp.dot(inv_ref[...], r_ref[...], preferred_element_type=_f32)
  z = (o_ref[0] + o_ref[1]) * inv64 + b1_ref[...]
  z = jnp.where(z > 0, z, jnp.exp(jnp.minimum(z, 0.0)) - 1.0)
  ri = lax.broadcasted_iota(jnp.int32, (NP, 64), 0)
  z = jnp.where(ri < N, z, 0.0)
  h2 = jnp.dot(z, w2_ref[...], preferred_element_type=_f32)
  h2_ref[...] = h2
  a2s = jnp.dot(h2, as2_ref[...], preferred_element_type=_f32)
  a2d = jnp.dot(h2, ad2_ref[...], preferred_element_type=_f32)
  a2s_ref[...] = a2s
  a2d_ref[...] = a2d
  g2_ref[...] = (jnp.max(a2s, axis=0) + jnp.max(a2d, axis=0)).reshape(1, L)


def _tc3_body(o_ref, b2_ref, out_ref):
  o = o_ref[0] + o_ref[1] + b2_ref[...]
  ri = lax.broadcasted_iota(jnp.int32, (NP, OUT), 0)
  o = jnp.where(ri < N, o, 0.0)
  m = jnp.max(o, axis=1, keepdims=True)
  o = o - m
  out_ref[...] = o - jnp.log(jnp.sum(jnp.exp(o), axis=1, keepdims=True))


def kernel(node_feature, adj_list, W1, att_src1, att_dst1, b1,
           W2, att_src2, att_dst2, b2):
  W1r = W1.reshape(IN, HEADS, MID)
  AS1 = jnp.pad(jnp.einsum("ihm,hm->ih", W1r, att_src1),
                ((0, 0), (0, L - HEADS)))
  AD1 = jnp.pad(jnp.einsum("ihm,hm->ih", W1r, att_dst1),
                ((0, 0), (0, L - HEADS)))
  Wcat = jnp.concatenate([W1, AS1, AD1], axis=1)

  eye8 = jnp.eye(HEADS, dtype=_f32)
  AS2 = jnp.pad((att_src2[:, :, None] * eye8[:, None, :])
                .reshape(HEADS * OUT, HEADS), ((0, 0), (0, L - HEADS)))
  AD2 = jnp.pad((att_dst2[:, :, None] * eye8[:, None, :])
                .reshape(HEADS * OUT, HEADS), ((0, 0), (0, L - HEADS)))

  xp = jnp.pad(node_feature, ((0, NP - N), (0, 0)))
  pad_e = jnp.full((EP - E,), N, jnp.int32)
  srcp = jnp.concatenate([adj_list[0].astype(jnp.int32), pad_e])
  dstp = jnp.concatenate([adj_list[1].astype(jnp.int32), pad_e])
  srcA = srcp.reshape(EP // L, L)
  dstA = dstp.reshape(EP // L, L)

  R16 = jnp.pad(jnp.repeat(jnp.eye(HEADS, dtype=_f32), MID, axis=1),
                ((0, L - HEADS), (0, 0)))
  z16 = jnp.zeros((CA, L), _f32)
  z4 = jnp.zeros((CA, 4, L), _f32)

  h1t, a1s, a1d, g1 = pl.pallas_call(
      _tc1_body,
      out_shape=[jax.ShapeDtypeStruct((NP, 64), _f32),
                 jax.ShapeDtypeStruct((NP, L), _f32),
                 jax.ShapeDtypeStruct((NP, L), _f32),
                 jax.ShapeDtypeStruct((1, L), _f32)])(xp, Wcat)

  ea1, den1 = _passA()(srcA, dstA, a1s, a1d, g1.reshape(L), z16)
  inv1 = pl.pallas_call(
      functools.partial(_dsum_body, 1.0),
      out_shape=jax.ShapeDtypeStruct((NP, L), _f32))(den1)
  (o1,) = _passB1()(srcA, dstA, ea1, h1t.reshape(NP, 4, L), z4)

  h2t, a2s, a2d, g2 = pl.pallas_call(
      _tc2_body,
      out_shape=[jax.ShapeDtypeStruct((NP, HEADS * OUT), _f32),
                 jax.ShapeDtypeStruct((NP, L), _f32),
                 jax.ShapeDtypeStruct((NP, L), _f32),
                 jax.ShapeDtypeStruct((1, L), _f32)])(
                     o1.reshape(NC, NP, 64), inv1, R16,
                     b1.reshape(1, 64), W2, AS2, AD2)

  ea2, den2 = _passA()(srcA, dstA, a2s, a2d, g2.reshape(L), z16)
  inv2 = pl.pallas_call(
      functools.partial(_dsum_body, 1.0 / HEADS),
      out_shape=jax.ShapeDtypeStruct((NP, L), _f32))(den2)
  h2q = lax.bitcast_convert_type(
      h2t.reshape(NP, L, 2, L).swapaxes(2, 3).astype(jnp.bfloat16),
      jnp.int32)
  (o2,) = _passB2()(srcp, dstp, ea2, inv2, h2q.reshape(NP, L, L), z4)

  outp = pl.pallas_call(
      _tc3_body,
      out_shape=jax.ShapeDtypeStruct((NP, OUT), _f32))(
          o2.reshape(NC, NP, OUT), b2.reshape(1, OUT))
  return outp[:N]


# speedup vs baseline: 1.1421x; 1.0017x over previous
import functools

import jax
import jax.numpy as jnp
from jax import lax
from jax.experimental import pallas as pl
from jax.experimental.pallas import tpu as pltpu
from jax.experimental.pallas import tpu_sc as plsc

N = 10000
E = 320000
HEADS = 8
MID = 8
OUT = 64
IN = 128
SLOPE = 0.28

NC = 2
NS = 16
NW = NC * NS
L = 16

NP = 10112
EP = 327680
CA = 128
CB2 = 64
RPT = NP // NS

NORTH = 1
ETN = 10240
ETS = 10240
NT = NS * ETN

_f32 = jnp.float32

_GDN = lax.GatherDimensionNumbers(
    offset_dims=(), collapsed_slice_dims=(0,), start_index_map=(0,))


def _lgather(v, idx):
  return lax.gather(v, idx.reshape(L, 1), _GDN, (1,),
                    mode=lax.GatherScatterMode.PROMISE_IN_BOUNDS)


def _bcast(v, lane):
  return _lgather(v, jnp.full((L,), lane, jnp.int32))


def _sc_mesh():
  return plsc.VectorSubcoreMesh(
      core_axis_name="c", subcore_axis_name="s", num_cores=NC,
      num_subcores=NS)


def _passA_body(src_h, dst_h, as_h, ad_h, g_h, z16_h, ea_h, den_h,
                idxs, idxd, is0, id0, is1, id1, ids0, ids1,
                asb0, adb0, asb1, adb1, eab0, eab1, gbuf,
                gsem0, gsem1, wsem0, wsem1, den_sp):
  cid = lax.axis_index("c")
  sid = lax.axis_index("s")
  row0 = pl.multiple_of(sid * RPT, 8)
  for i in range(RPT // CA):
    pltpu.sync_copy(z16_h, den_sp.at[pl.ds(row0 + i * CA, CA)])
  pltpu.sync_copy(z16_h.at[pl.ds(0, RPT % CA)],
                  den_sp.at[pl.ds(row0 + (RPT // CA) * CA, RPT % CA)])
  pltpu.sync_copy(g_h, gbuf)
  plsc.subcore_barrier()
  gv = gbuf[...]

  def ldidx(src2d, k, dst1d):
    for j in range(CA // L):
      dst1d[pl.ds(j * L, L)] = src2d[k * (CA // L) + j]

  def pipe(core_base, ett):
    er = ett // L
    rb = pl.multiple_of((core_base + sid * ett) // L, 8)
    pltpu.sync_copy(src_h.at[pl.ds(rb, er)], idxs.at[pl.ds(0, er)])
    pltpu.sync_copy(dst_h.at[pl.ds(rb, er)], idxd.at[pl.ds(0, er)])
    base0 = core_base + sid * ett

    def issue(k, isb, idb_, asb, adb, sem):
      ldidx(idxs, k, isb)
      ldidx(idxd, k, idb_)
      pltpu.async_copy(as_h.at[isb], asb, sem)
      pltpu.async_copy(ad_h.at[idb_], adb, sem)

    def wait_g(isb, idb_, asb, adb, sem):
      pltpu.make_async_copy(as_h.at[isb], asb, sem).wait()
      pltpu.make_async_copy(ad_h.at[idb_], adb, sem).wait()

    def compute(k, sidb, asb, adb, eab, wsem):
      def edge(e, cc):
        v = asb[e] + adb[e]
        a = jnp.maximum(v, v * SLOPE) - gv
        eab[e] = jnp.exp(a)
        return cc
      lax.fori_loop(0, CA, edge, 0, unroll=4)
      base = pl.multiple_of(base0 + k * CA, CA)
      pltpu.async_copy(eab, ea_h.at[pl.ds(base, CA)], wsem)
      ldidx(idxd, k, sidb)
      pltpu.sync_copy(eab, den_sp.at[sidb], add=True)

    def wait_w(eab, wsem):
      pltpu.make_async_copy(eab, ea_h.at[pl.ds(0, CA)], wsem).wait()

    nh = ett // CA // 2
    issue(0, is0, id0, asb0, adb0, gsem0)

    def body(k2, cc):
      k = 2 * k2
      issue(k + 1, is1, id1, asb1, adb1, gsem1)
      wait_g(is0, id0, asb0, adb0, gsem0)

      @pl.when(k2 > 0)
      def _():
        wait_w(eab0, wsem0)
      compute(k, ids0, asb0, adb0, eab0, wsem0)

      @pl.when(k2 < nh - 1)
      def _():
        issue(k + 2, is0, id0, asb0, adb0, gsem0)
      wait_g(is1, id1, asb1, adb1, gsem1)

      @pl.when(k2 > 0)
      def _():
        wait_w(eab1, wsem1)
      compute(k + 1, ids1, asb1, adb1, eab1, wsem1)
      return cc

    lax.fori_loop(0, nh, body, 0)
    wait_w(eab0, wsem0)
    wait_w(eab1, wsem1)

  @pl.when(cid == NORTH)
  def _():
    pipe(0, ETN)

  @pl.when(cid != NORTH)
  def _():
    pipe(NT, ETS)

  plsc.subcore_barrier()
  pltpu.sync_copy(den_sp.at[pl.ds(row0, RPT)],
                  den_h.at[cid, pl.ds(row0, RPT)])


_passA = functools.partial(
    pl.kernel, _passA_body,
    out_type=[jax.ShapeDtypeStruct((EP, L), _f32),
              jax.ShapeDtypeStruct((NC, NP, L), _f32)],
    mesh=_sc_mesh(),
    compiler_params=pltpu.CompilerParams(use_tc_tiling_on_sc=False),
    scratch_types=[pltpu.VMEM((ETN // L, L), jnp.int32),
                   pltpu.VMEM((ETN // L, L), jnp.int32),
                   pltpu.VMEM((CA,), jnp.int32),
                   pltpu.VMEM((CA,), jnp.int32),
                   pltpu.VMEM((CA,), jnp.int32),
                   pltpu.VMEM((CA,), jnp.int32),
                   pltpu.VMEM((CA,), jnp.int32),
                   pltpu.VMEM((CA,), jnp.int32),
                   pltpu.VMEM((CA, L), _f32),
                   pltpu.VMEM((CA, L), _f32),
                   pltpu.VMEM((CA, L), _f32),
                   pltpu.VMEM((CA, L), _f32),
                   pltpu.VMEM((CA, L), _f32),
                   pltpu.VMEM((CA, L), _f32),
                   pltpu.VMEM((L,), _f32),
                   pltpu.SemaphoreType.DMA,
                   pltpu.SemaphoreType.DMA,
                   pltpu.SemaphoreType.DMA,
                   pltpu.SemaphoreType.DMA,
                   pltpu.VMEM_SHARED((NP, L), _f32)])


def _passB1_body(src_h, dst_h, ea_h, htab_h, z4_h, o_h,
                 idxs, idxd, is0, id0, is1, id1, ids0, ids1,
                 hb0, eab0, hb1, eab1, msg0, msg1,
                 gsem0, gsem1, ssem0, ssem1, out_sp):
  cid = lax.axis_index("c")
  sid = lax.axis_index("s")
  row0 = pl.multiple_of(sid * RPT, 8)
  for i in range(RPT // CA):
    pltpu.sync_copy(z4_h, out_sp.at[pl.ds(row0 + i * CA, CA)])
  pltpu.sync_copy(z4_h.at[pl.ds(0, RPT % CA)],
                  out_sp.at[pl.ds(row0 + (RPT // CA) * CA, RPT % CA)])
  plsc.subcore_barrier()
  half = lax.shift_right_logical(
      lax.broadcasted_iota(jnp.int32, (L,), 0), 3)

  def ldidx(src2d, k, dst1d):
    for j in range(CA // L):
      dst1d[pl.ds(j * L, L)] = src2d[k * (CA // L) + j]

  def pipe(core_base, ett):
    er = ett // L
    rb = pl.multiple_of((core_base + sid * ett) // L, 8)
    pltpu.sync_copy(src_h.at[pl.ds(rb, er)], idxs.at[pl.ds(0, er)])
    pltpu.sync_copy(dst_h.at[pl.ds(rb, er)], idxd.at[pl.ds(0, er)])
    base0 = core_base + sid * ett

    def issue(k, isb, hb, eab, sem):
      ldidx(idxs, k, isb)
      pltpu.async_copy(htab_h.at[isb], hb, sem)
      base = pl.multiple_of(base0 + k * CA, CA)
      pltpu.async_copy(ea_h.at[pl.ds(base, CA)], eab, sem)

    def wait_g(isb, hb, eab, sem):
      pltpu.make_async_copy(htab_h.at[isb], hb, sem).wait()
      pltpu.make_async_copy(ea_h.at[pl.ds(0, CA)], eab, sem).wait()

    def compute(k, sidb, hb, eab, msg, ssem):
      ldidx(idxd, k, sidb)

      def edge(e, cc):
        cv = eab[e]
        for j in range(4):
          cj = _lgather(cv, 2 * j + half)
          msg[e, j] = hb[e, j] * cj
        return cc
      lax.fori_loop(0, CA, edge, 0, unroll=4)
      pltpu.async_copy(msg, out_sp.at[sidb], ssem, add=True)

    def wait_s(sidb, msg, ssem):
      pltpu.make_async_copy(msg, out_sp.at[sidb], ssem).wait()

    nh = ett // CA // 2
    issue(0, is0, hb0, eab0, gsem0)

    def body(k2, cc):
      k = 2 * k2
      issue(k + 1, is1, hb1, eab1, gsem1)
      wait_g(is0, hb0, eab0, gsem0)

      @pl.when(k2 > 0)
      def _():
        wait_s(ids0, msg0, ssem0)
      compute(k, ids0, hb0, eab0, msg0, ssem0)

      @pl.when(k2 < nh - 1)
      def _():
        issue(k + 2, is0, hb0, eab0, gsem0)
      wait_g(is1, hb1, eab1, gsem1)

      @pl.when(k2 > 0)
      def _():
        wait_s(ids1, msg1, ssem1)
      compute(k + 1, ids1, hb1, eab1, msg1, ssem1)
      return cc

    lax.fori_loop(0, nh, body, 0)
    wait_s(ids0, msg0, ssem0)
    wait_s(ids1, msg1, ssem1)

  @pl.when(cid == NORTH)
  def _():
    pipe(0, ETN)

  @pl.when(cid != NORTH)
  def _():
    pipe(NT, ETS)

  plsc.subcore_barrier()
  pltpu.sync_copy(out_sp.at[pl.ds(row0, RPT)],
                  o_h.at[cid, pl.ds(row0, RPT)])


_passB1 = functools.partial(
    pl.kernel, _passB1_body,
    out_type=[jax.ShapeDtypeStruct((NC, NP, 4, L), _f32)],
    mesh=_sc_mesh(),
    compiler_params=pltpu.CompilerParams(use_tc_tiling_on_sc=False),
    scratch_types=[pltpu.VMEM((ETN // L, L), jnp.int32),
                   pltpu.VMEM((ETN // L, L), jnp.int32),
                   pltpu.VMEM((CA,), jnp.int32),
                   pltpu.VMEM((CA,), jnp.int32),
                   pltpu.VMEM((CA,), jnp.int32),
                   pltpu.VMEM((CA,), jnp.int32),
                   pltpu.VMEM((CA,), jnp.int32),
                   pltpu.VMEM((CA,), jnp.int32),
                   pltpu.VMEM((CA, 4, L), _f32),
                   pltpu.VMEM((CA, L), _f32),
                   pltpu.VMEM((CA, 4, L), _f32),
                   pltpu.VMEM((CA, L), _f32),
                   pltpu.VMEM((CA, 4, L), _f32),
                   pltpu.VMEM((CA, 4, L), _f32),
                   pltpu.SemaphoreType.DMA,
                   pltpu.SemaphoreType.DMA,
                   pltpu.SemaphoreType.DMA,
                   pltpu.SemaphoreType.DMA,
                   pltpu.VMEM_SHARED((NP, 4, L), _f32)])


def _passB2_body(src_h, dst_h, ea_h, inv_h, htab_h, z4_h, o_h,
                 is0, id0, is1, id1, ids0, ids1,
                 hb0, idb0, eab0, hb1, idb1, eab1, msg0, msg1,
                 gsem0, gsem1, ssem0, ssem1, isem0, isem1, out_sp):
  cid = lax.axis_index("c")
  sid = lax.axis_index("s")
  row0 = pl.multiple_of(sid * RPT, 8)
  for i in range(RPT // CA):
    pltpu.sync_copy(z4_h, out_sp.at[pl.ds(row0 + i * CA, CA)])
  pltpu.sync_copy(z4_h.at[pl.ds(0, RPT % CA)],
                  out_sp.at[pl.ds(row0 + (RPT // CA) * CA, RPT % CA)])
  plsc.subcore_barrier()

  def cpidx(idb_, sidb):
    for j in range(CB2 // L):
      sidb[pl.ds(j * L, L)] = idb_[pl.ds(j * L, L)]

  def pipe(core_base, ett):
    base0 = core_base + sid * ett

    def issue_idx(k, isb, idb_, isem):
      base = pl.multiple_of(base0 + k * CB2, CB2)
      pltpu.async_copy(src_h.at[pl.ds(base, CB2)], isb, isem)
      pltpu.async_copy(dst_h.at[pl.ds(base, CB2)], idb_, isem)

    def wait_idx(isb, idb_, isem):
      pltpu.make_async_copy(src_h.at[pl.ds(0, CB2)], isb, isem).wait()
      pltpu.make_async_copy(dst_h.at[pl.ds(0, CB2)], idb_, isem).wait()

    def issue_g(k, isb, idb_, hb, idb, eab, sem):
      pltpu.async_copy(htab_h.at[isb], hb, sem)
      pltpu.async_copy(inv_h.at[idb_], idb, sem)
      base = pl.multiple_of(base0 + k * CB2, CB2)
      pltpu.async_copy(ea_h.at[pl.ds(base, CB2)], eab, sem)

    def wait_g(isb, idb_, hb, idb, eab, sem):
      pltpu.make_async_copy(htab_h.at[isb], hb, sem).wait()
      pltpu.make_async_copy(inv_h.at[idb_], idb, sem).wait()
      pltpu.make_async_copy(ea_h.at[pl.ds(0, CB2)], eab, sem).wait()

    def compute(hb, idb, eab, msg, sidb, ssem):
      def edge(e, cc):
        cv = eab[e] * idb[e]
        acc = [None] * 4
        for h in range(HEADS):
          cf = _bcast(cv, h)
          for p in range(2):
            w = hb[e, 2 * h + p]
            lo = lax.bitcast_convert_type(
                lax.shift_left(w, jnp.int32(16)), _f32)
            hi = lax.bitcast_convert_type(
                lax.bitwise_and(w, jnp.int32(-65536)), _f32)
            for q, g in ((0, lo), (1, hi)):
              j = 2 * p + q
              t = g * cf
              acc[j] = t if acc[j] is None else acc[j] + t
        for j in range(4):
          msg[e, j] = acc[j]
        return cc
      lax.fori_loop(0, CB2, edge, 0, unroll=4)
      pltpu.async_copy(msg, out_sp.at[sidb], ssem, add=True)

    def wait_s(sidb, msg, ssem):
      pltpu.make_async_copy(msg, out_sp.at[sidb], ssem).wait()

    nh = ett // CB2 // 2
    issue_idx(0, is0, id0, isem0)
    issue_idx(1, is1, id1, isem1)
    wait_idx(is0, id0, isem0)
    issue_g(0, is0, id0, hb0, idb0, eab0, gsem0)

    def body(k2, cc):
      k = 2 * k2
      wait_idx(is1, id1, isem1)
      issue_g(k + 1, is1, id1, hb1, idb1, eab1, gsem1)

      wait_g(is0, id0, hb0, idb0, eab0, gsem0)

      @pl.when(k2 > 0)
      def _():
        wait_s(ids0, msg0, ssem0)
      cpidx(id0, ids0)

      @pl.when(k2 < nh - 1)
      def _():
        issue_idx(k + 2, is0, id0, isem0)
      compute(hb0, idb0, eab0, msg0, ids0, ssem0)

      @pl.when(k2 < nh - 1)
      def _():
        wait_idx(is0, id0, isem0)
        issue_g(k + 2, is0, id0, hb0, idb0, eab0, gsem0)

      wait_g(is1, id1, hb1, idb1, eab1, gsem1)

      @pl.when(k2 > 0)
      def _():
        wait_s(ids1, msg1, ssem1)
      cpidx(id1, ids1)

      @pl.when(k2 < nh - 1)
      def _():
        issue_idx(k + 3, is1, id1, isem1)
      compute(hb1, idb1, eab1, msg1, ids1, ssem1)
      return cc

    lax.fori_loop(0, nh, body, 0)
    wait_s(ids0, msg0, ssem0)
    wait_s(ids1, msg1, ssem1)

  @pl.when(cid == NORTH)
  def _():
    pipe(0, ETN)

  @pl.when(cid != NORTH)
  def _():
    pipe(NT, ETS)

  plsc.subcore_barrier()
  pltpu.sync_copy(out_sp.at[pl.ds(row0, RPT)],
                  o_h.at[cid, pl.ds(row0, RPT)])


_passB2 = functools.partial(
    pl.kernel, _passB2_body,
    out_type=[jax.ShapeDtypeStruct((NC, NP, 4, L), _f32)],
    mesh=_sc_mesh(),
    compiler_params=pltpu.CompilerParams(use_tc_tiling_on_sc=False),
    scratch_types=[pltpu.VMEM((CB2,), jnp.int32),
                   pltpu.VMEM((CB2,), jnp.int32),
                   pltpu.VMEM((CB2,), jnp.int32),
                   pltpu.VMEM((CB2,), jnp.int32),
                   pltpu.VMEM((CB2,), jnp.int32),
                   pltpu.VMEM((CB2,), jnp.int32),
                   pltpu.VMEM((CB2, 16, L), jnp.int32),
                   pltpu.VMEM((CB2, L), _f32),
                   pltpu.VMEM((CB2, L), _f32),
                   pltpu.VMEM((CB2, 16, L), jnp.int32),
                   pltpu.VMEM((CB2, L), _f32),
                   pltpu.VMEM((CB2, L), _f32),
                   pltpu.VMEM((CB2, 4, L), _f32),
                   pltpu.VMEM((CB2, 4, L), _f32),
                   pltpu.SemaphoreType.DMA,
                   pltpu.SemaphoreType.DMA,
                   pltpu.SemaphoreType.DMA,
                   pltpu.SemaphoreType.DMA,
                   pltpu.SemaphoreType.DMA,
                   pltpu.SemaphoreType.DMA,
                   pltpu.VMEM_SHARED((NP, 4, L), _f32)])


def _tc1_body(xp_ref, w_ref, h_ref, as_ref, ad_ref, g_ref):
  hcat = jnp.dot(xp_ref[...], w_ref[...], preferred_element_type=_f32)
  h_ref[...] = hcat[:, :64]
  a_s = hcat[:, 64:80]
  a_d = hcat[:, 80:96]
  as_ref[...] = a_s
  ad_ref[...] = a_d
  g_ref[...] = (jnp.max(a_s, axis=0) + jnp.max(a_d, axis=0)).reshape(1, L)


def _dsum_body(scale, den_ref, inv_ref):
  d = den_ref[0] + den_ref[1]
  inv_ref[...] = scale / (d + 1e-16)


def _tc2_body(o_ref, inv_ref, r_ref, b1_ref, w2_ref, as2_ref, ad2_ref,
              h2_ref, a2s_ref, a2d_ref, g2_ref):
  inv64 = jnp.dot(inv_ref[...], r_ref[...], preferred_element_type=_f32)
  z = (o_ref[0] + o_ref[1]) * inv64 + b1_ref[...]
  z = jnp.where(z > 0, z, jnp.exp(jnp.minimum(z, 0.0)) - 1.0)
  ri = lax.broadcasted_iota(jnp.int32, (NP, 64), 0)
  z = jnp.where(ri < N, z, 0.0)
  h2 = jnp.dot(z, w2_ref[...], preferred_element_type=_f32)
  h2_ref[...] = h2
  a2s = jnp.dot(h2, as2_ref[...], preferred_element_type=_f32)
  a2d = jnp.dot(h2, ad2_ref[...], preferred_element_type=_f32)
  a2s_ref[...] = a2s
  a2d_ref[...] = a2d
  g2_ref[...] = (jnp.max(a2s, axis=0) + jnp.max(a2d, axis=0)).reshape(1, L)


def _tc3_body(o_ref, b2_ref, out_ref):
  o = o_ref[0] + o_ref[1] + b2_ref[...]
  ri = lax.broadcasted_iota(jnp.int32, (NP, OUT), 0)
  o = jnp.where(ri < N, o, 0.0)
  m = jnp.max(o, axis=1, keepdims=True)
  o = o - m
  out_ref[...] = o - jnp.log(jnp.sum(jnp.exp(o), axis=1, keepdims=True))


def kernel(node_feature, adj_list, W1, att_src1, att_dst1, b1,
           W2, att_src2, att_dst2, b2):
  W1r = W1.reshape(IN, HEADS, MID)
  AS1 = jnp.pad(jnp.einsum("ihm,hm->ih", W1r, att_src1),
                ((0, 0), (0, L - HEADS)))
  AD1 = jnp.pad(jnp.einsum("ihm,hm->ih", W1r, att_dst1),
                ((0, 0), (0, L - HEADS)))
  Wcat = jnp.concatenate([W1, AS1, AD1], axis=1)

  eye8 = jnp.eye(HEADS, dtype=_f32)
  AS2 = jnp.pad((att_src2[:, :, None] * eye8[:, None, :])
                .reshape(HEADS * OUT, HEADS), ((0, 0), (0, L - HEADS)))
  AD2 = jnp.pad((att_dst2[:, :, None] * eye8[:, None, :])
                .reshape(HEADS * OUT, HEADS), ((0, 0), (0, L - HEADS)))

  xp = jnp.pad(node_feature, ((0, NP - N), (0, 0)))
  pad_e = jnp.full((EP - E,), N, jnp.int32)
  srcp = jnp.concatenate([adj_list[0].astype(jnp.int32), pad_e])
  dstp = jnp.concatenate([adj_list[1].astype(jnp.int32), pad_e])
  srcA = srcp.reshape(EP // L, L)
  dstA = dstp.reshape(EP // L, L)

  R16 = jnp.pad(jnp.repeat(jnp.eye(HEADS, dtype=_f32), MID, axis=1),
                ((0, L - HEADS), (0, 0)))
  z16 = jnp.zeros((CA, L), _f32)
  z4 = jnp.zeros((CA, 4, L), _f32)

  h1t, a1s, a1d, g1 = pl.pallas_call(
      _tc1_body,
      out_shape=[jax.ShapeDtypeStruct((NP, 64), _f32),
                 jax.ShapeDtypeStruct((NP, L), _f32),
                 jax.ShapeDtypeStruct((NP, L), _f32),
                 jax.ShapeDtypeStruct((1, L), _f32)])(xp, Wcat)

  ea1, den1 = _passA()(srcA, dstA, a1s, a1d, g1.reshape(L), z16)
  inv1 = pl.pallas_call(
      functools.partial(_dsum_body, 1.0),
      out_shape=jax.ShapeDtypeStruct((NP, L), _f32))(den1)
  (o1,) = _passB1()(srcA, dstA, ea1, h1t.reshape(NP, 4, L), z4)

  h2t, a2s, a2d, g2 = pl.pallas_call(
      _tc2_body,
      out_shape=[jax.ShapeDtypeStruct((NP, HEADS * OUT), _f32),
                 jax.ShapeDtypeStruct((NP, L), _f32),
                 jax.ShapeDtypeStruct((NP, L), _f32),
                 jax.ShapeDtypeStruct((1, L), _f32)])(
                     o1.reshape(NC, NP, 64), inv1, R16,
                     b1.reshape(1, 64), W2, AS2, AD2)

  ea2, den2 = _passA()(srcA, dstA, a2s, a2d, g2.reshape(L), z16)
  inv2 = pl.pallas_call(
      functools.partial(_dsum_body, 1.0 / HEADS),
      out_shape=jax.ShapeDtypeStruct((NP, L), _f32))(den2)
  h2q = lax.bitcast_convert_type(
      h2t.reshape(NP, L, 2, L).swapaxes(2, 3).astype(jnp.bfloat16),
      jnp.int32)
  (o2,) = _passB2()(srcp, dstp, ea2, inv2, h2q.reshape(NP, L, L), z4)

  outp = pl.pallas_call(
      _tc3_body,
      out_shape=jax.ShapeDtypeStruct((NP, OUT), _f32))(
          o2.reshape(NC, NP, OUT), b2.reshape(1, OUT))
  return outp[:N]
